# Initial kernel scaffold; baseline (speedup 1.0000x reference)
import jax
import jax.numpy as jnp
from jax.experimental import pallas as pl


def kernel(x, edge_index, batch, sigma, W1, b1, W2, b2, W3, b3, g1, be1, g2, be2, g3, be3, Wf1, bf1, Wf2, bf2, Wfc, bfc):
    raise NotImplementedError("write your pallas kernel here")


# speedup vs baseline: 16.0191x; 16.0191x over previous
import functools

import jax
import jax.numpy as jnp
from jax import lax
from jax.experimental import pallas as pl
from jax.experimental.pallas import tpu as pltpu
from jax.experimental.pallas import tpu_sc as plsc

_K = 128


def _mesh():
    return plsc.VectorSubcoreMesh(core_axis_name="c", subcore_axis_name="s")


@functools.lru_cache(maxsize=None)
def _deg_call(n, e, nc, ns):
    nw = nc * ns
    ew = e // nw
    assert ew * nw == e
    nfull = ew // _K
    tail = ew - nfull * _K
    assert tail % 8 == 0
    rpt = n // ns
    assert rpt * ns == n
    sz = -((rpt + 7) // -8) * 8
    assert ((ns - 1) * rpt // 8) * 8 + sz == n

    @functools.partial(
        pl.kernel,
        out_type=jax.ShapeDtypeStruct((nc, n, 16), jnp.float32),
        mesh=_mesh(),
        compiler_params=pltpu.CompilerParams(use_tc_tiling_on_sc=False),
        scratch_types=[
            pltpu.VMEM((_K,), jnp.int32),
            pltpu.VMEM((max(tail, 8),), jnp.int32),
            pltpu.VMEM((_K, 16), jnp.float32),
            pltpu.VMEM((sz, 16), jnp.float32),
            pltpu.VMEM_SHARED((n, 16), jnp.float32),
        ],
    )
    def deg_kernel(dst_hbm, ones_hbm, zeros_hbm, out_hbm,
                   idx_v, idx_t_v, ones_v, stage_v, acc_sh):
        c = lax.axis_index("c")
        s = lax.axis_index("s")
        w = c * ns + s
        start = pl.multiple_of(s * rpt // 8 * 8, 8)
        pltpu.sync_copy(ones_hbm, ones_v)
        pltpu.sync_copy(zeros_hbm.at[pl.ds(start, sz)], stage_v)
        pltpu.sync_copy(stage_v, acc_sh.at[pl.ds(start, sz)])
        plsc.subcore_barrier()
        base = w * ew

        def body(j, carry):
            off = base + j * _K
            pltpu.sync_copy(dst_hbm.at[pl.ds(off, _K)], idx_v)
            pltpu.sync_copy(ones_v, acc_sh.at[idx_v], add=True)
            return carry

        lax.fori_loop(0, nfull, body, jnp.int32(0))
        if tail:
            off = base + nfull * _K
            pltpu.sync_copy(dst_hbm.at[pl.ds(off, tail)], idx_t_v)
            pltpu.sync_copy(ones_v.at[pl.ds(0, tail)],
                            acc_sh.at[idx_t_v], add=True)
        plsc.subcore_barrier()
        pltpu.sync_copy(acc_sh.at[pl.ds(start, sz)], stage_v)
        pltpu.sync_copy(stage_v, out_hbm.at[c, pl.ds(start, sz)])

    return deg_kernel


@functools.lru_cache(maxsize=None)
def _agg_call(n, h, e, nc, ns):
    nw = nc * ns
    ew = e // nw
    assert ew * nw == e
    nfull = ew // _K
    tail = ew - nfull * _K
    assert tail % 8 == 0
    rpt = n // ns
    assert rpt * ns == n
    sz = -((rpt + 7) // -8) * 8
    assert ((ns - 1) * rpt // 8) * 8 + sz == n

    @functools.partial(
        pl.kernel,
        out_type=jax.ShapeDtypeStruct((nc, n, h), jnp.float32),
        mesh=_mesh(),
        compiler_params=pltpu.CompilerParams(use_tc_tiling_on_sc=False),
        scratch_types=[
            pltpu.VMEM((_K,), jnp.int32),
            pltpu.VMEM((_K,), jnp.int32),
            pltpu.VMEM((max(tail, 8),), jnp.int32),
            pltpu.VMEM((max(tail, 8),), jnp.int32),
            pltpu.VMEM((_K, h), jnp.float32),
            pltpu.VMEM((max(tail, 8), h), jnp.float32),
            pltpu.VMEM((sz, h), jnp.float32),
            pltpu.VMEM_SHARED((n, h), jnp.float32),
            pltpu.SemaphoreType.DMA,
        ],
    )
    def agg_kernel(src_hbm, dst_hbm, s_hbm, zeros_hbm, out_hbm,
                   idx_s_v, idx_d_v, idx_st_v, idx_dt_v,
                   rows_v, rows_t_v, stage_v, acc_sh, sem):
        c = lax.axis_index("c")
        s = lax.axis_index("s")
        w = c * ns + s
        start = pl.multiple_of(s * rpt // 8 * 8, 8)
        pltpu.sync_copy(zeros_hbm.at[pl.ds(start, sz)], stage_v)
        pltpu.sync_copy(stage_v, acc_sh.at[pl.ds(start, sz)])
        plsc.subcore_barrier()
        base = w * ew

        def body(j, carry):
            off = base + j * _K
            pltpu.sync_copy(src_hbm.at[pl.ds(off, _K)], idx_s_v)
            pltpu.sync_copy(dst_hbm.at[pl.ds(off, _K)], idx_d_v)
            pltpu.async_copy(s_hbm.at[idx_s_v], rows_v, sem).wait()
            pltpu.sync_copy(rows_v, acc_sh.at[idx_d_v], add=True)
            return carry

        lax.fori_loop(0, nfull, body, jnp.int32(0))
        if tail:
            off = base + nfull * _K
            pltpu.sync_copy(src_hbm.at[pl.ds(off, tail)], idx_st_v)
            pltpu.sync_copy(dst_hbm.at[pl.ds(off, tail)], idx_dt_v)
            pltpu.async_copy(s_hbm.at[idx_st_v], rows_t_v, sem).wait()
            pltpu.sync_copy(rows_t_v, acc_sh.at[idx_dt_v], add=True)
        plsc.subcore_barrier()
        pltpu.sync_copy(acc_sh.at[pl.ds(start, sz)], stage_v)
        pltpu.sync_copy(stage_v, out_hbm.at[c, pl.ds(start, sz)])

    return agg_kernel


def _dot(a, b):
    return jnp.dot(a, b, precision=jax.lax.Precision.HIGHEST,
                   preferred_element_type=jnp.float32)


def _bn_norm(agg, g, be):
    m = jnp.mean(agg, axis=0, keepdims=True)
    v = jnp.mean((agg - m) ** 2, axis=0, keepdims=True)
    return (agg - m) / jnp.sqrt(v + 1e-5) * g + be


def _tc1_body(degp_ref, x_ref, w1_ref, dinv_ref, s1_ref):
    deg = 1.0 + degp_ref[0, :, 0:1] + degp_ref[1, :, 0:1]
    dinv = lax.rsqrt(deg)
    dinv_ref[...] = dinv
    s1_ref[...] = _dot(x_ref[...], w1_ref[...]) * dinv


def _tc2_body(t1_ref, s1_ref, dinv_ref, b1_ref, g1_ref, be1_ref, w2_ref,
              h1_ref, s2_ref):
    dinv = dinv_ref[...]
    agg = dinv * (t1_ref[0] + t1_ref[1] + s1_ref[...]) + b1_ref[...]
    h1 = jnp.maximum(_bn_norm(agg, g1_ref[...], be1_ref[...]), 0.0)
    h1_ref[...] = h1
    s2_ref[...] = _dot(h1, w2_ref[...]) * dinv


def _tc3_body(t2_ref, s2_ref, dinv_ref, b2_ref, g2_ref, be2_ref, h1_ref,
              w3_ref, s3_ref):
    dinv = dinv_ref[...]
    agg = dinv * (t2_ref[0] + t2_ref[1] + s2_ref[...]) + b2_ref[...]
    h2 = jnp.maximum(_bn_norm(agg, g2_ref[...], be2_ref[...]) + h1_ref[...],
                     0.0)
    s3_ref[...] = _dot(h2, w3_ref[...]) * dinv


def _tc4_body(t3_ref, s3_ref, dinv_ref, b3_ref, g3_ref, be3_ref, batch_ref,
              sigma_ref, wf1_ref, bf1_ref, wf2_ref, bf2_ref, wfc_ref, bfc_ref,
              out_ref):
    gdim, n = out_ref.shape[0], s3_ref.shape[0]
    hdim = s3_ref.shape[1]
    dinv = dinv_ref[...]
    agg = dinv * (t3_ref[0] + t3_ref[1] + s3_ref[...]) + b3_ref[...]
    h3 = jnp.maximum(_bn_norm(agg, g3_ref[...], be3_ref[...]), 0.0)
    oh = (lax.broadcasted_iota(jnp.int32, (gdim, n), 0)
          == batch_ref[...]).astype(jnp.float32)
    sums = _dot(oh, h3)
    cnt = jnp.sum(oh, axis=1, keepdims=True)
    gemb = sums / jnp.maximum(cnt, 1.0)
    f = jnp.maximum(_dot(sigma_ref[...], wf1_ref[...]) + bf1_ref[...], 0.0)
    f = jnp.maximum(_dot(f, wf2_ref[...]) + bf2_ref[...], 0.0)
    out_ref[...] = (_dot(gemb, wfc_ref[0:hdim, :])
                    + _dot(f, wfc_ref[hdim:, :]) + bfc_ref[...])


def kernel(x, edge_index, batch, sigma, W1, b1, W2, b2, W3, b3,
           g1, be1, g2, be2, g3, be3, Wf1, bf1, Wf2, bf2, Wfc, bfc):
    n, d = x.shape
    h = W1.shape[1]
    g = sigma.shape[0]
    e = edge_index.shape[1]
    info = plsc.get_sparse_core_info()
    nc, ns = info.num_cores, info.num_subcores

    src = edge_index[0]
    dst = edge_index[1]
    zeros_h = jnp.zeros((n, h), jnp.float32)
    zeros16 = jnp.zeros((n, 16), jnp.float32)
    ones_k16 = jnp.ones((_K, 16), jnp.float32)

    degp = _deg_call(n, e, nc, ns)(dst, ones_k16, zeros16)

    f32 = jnp.float32
    dinv, s1 = pl.pallas_call(
        _tc1_body,
        out_shape=[jax.ShapeDtypeStruct((n, 1), f32),
                   jax.ShapeDtypeStruct((n, h), f32)],
    )(degp, x, W1)

    agg = _agg_call(n, h, e, nc, ns)
    t1 = agg(src, dst, s1, zeros_h)
    h1, s2 = pl.pallas_call(
        _tc2_body,
        out_shape=[jax.ShapeDtypeStruct((n, h), f32),
                   jax.ShapeDtypeStruct((n, h), f32)],
    )(t1, s1, dinv, b1, g1, be1, W2)

    t2 = agg(src, dst, s2, zeros_h)
    s3 = pl.pallas_call(
        _tc3_body,
        out_shape=jax.ShapeDtypeStruct((n, h), f32),
    )(t2, s2, dinv, b2, g2, be2, h1, W3)

    t3 = agg(src, dst, s3, zeros_h)
    out2d = pl.pallas_call(
        _tc4_body,
        out_shape=jax.ShapeDtypeStruct((g, 1), f32),
    )(t3, s3, dinv, b3, g3, be3, batch.reshape(1, n), sigma,
      Wf1, bf1, Wf2, bf2, Wfc, bfc)
    return out2d.reshape(g)
